# baseline (speedup 1.0000x reference)
import functools

import jax
import jax.numpy as jnp
from jax import lax
from jax.experimental import pallas as pl
from jax.experimental.pallas import tpu as pltpu
from jax.experimental.pallas import tpu_sc as plsc

B = 1024
L = 50
OUT_V = 1099
LANES = 16
NC = 2
NS = 16
NW = NC * NS
ROWS_PER_W = B // NW
GROUPS = ROWS_PER_W // LANES


@functools.partial(
    pl.kernel,
    mesh=plsc.VectorSubcoreMesh(core_axis_name="c", subcore_axis_name="s"),
    out_type=jax.ShapeDtypeStruct((B, OUT_V), jnp.float32),
    scratch_types=[
        pltpu.VMEM((ROWS_PER_W, L), jnp.int32),
        pltpu.VMEM((ROWS_PER_W, OUT_V), jnp.float32),
    ],
    compiler_params=pltpu.CompilerParams(
        use_tc_tiling_on_sc=False,
        needs_layout_passes=False,
        disable_bounds_checks=True,
        skip_device_barrier=True,
    ),
)
def _bag_of_words(in_hbm, out_hbm, tok_v, acc_v):
    wid = lax.axis_index("s") * NC + lax.axis_index("c")
    base = wid * ROWS_PER_W

    pltpu.sync_copy(in_hbm.at[pl.ds(base, ROWS_PER_W)], tok_v)

    lane = lax.iota(jnp.int32, LANES)
    zf = jnp.zeros((LANES,), jnp.float32)
    ones = jnp.ones((LANES,), jnp.float32)

    def zero_row(r, carry):
        for i in range(OUT_V // LANES):
            acc_v[r, pl.ds(i * LANES, LANES)] = zf
        tail_col = jnp.minimum(lane + (OUT_V // LANES) * LANES, OUT_V - 1)
        plsc.store_scatter(
            acc_v,
            [jnp.full((LANES,), 0, jnp.int32) + r, tail_col],
            zf,
            mask=lane < (OUT_V % LANES),
        )
        return carry

    lax.fori_loop(0, ROWS_PER_W, zero_row, 0)

    for g in range(GROUPS):
        row_idx = lane + (g * LANES)
        for l in range(L):
            col_l = jnp.full((LANES,), l, jnp.int32)
            tok = plsc.load_gather(tok_v, [row_idx, col_l])
            m = tok >= 1
            col = jnp.maximum(tok - 1, 0)
            plsc.addupdate_scatter(acc_v, [row_idx, col], ones, mask=m)

    pltpu.sync_copy(acc_v, out_hbm.at[pl.ds(base, ROWS_PER_W)])


def kernel(inputs):
    return _bag_of_words(inputs)


# speedup vs baseline: 1.0015x; 1.0015x over previous
import functools

import jax
import jax.numpy as jnp
from jax import lax
from jax.experimental import pallas as pl
from jax.experimental.pallas import tpu as pltpu
from jax.experimental.pallas import tpu_sc as plsc

B = 1024
L = 50
OUT_V = 1099
V_TILES = 9
V_PAD = V_TILES * 128
LANES = 16
NC = 2
NS = 16
NW = NC * NS
ROWS_PER_W = B // NW
RT_PER_W = ROWS_PER_W // 8
GROUPS = ROWS_PER_W // LANES


@functools.partial(
    pl.kernel,
    mesh=plsc.VectorSubcoreMesh(core_axis_name="c", subcore_axis_name="s"),
    out_type=jax.ShapeDtypeStruct((B // 8, V_TILES, 8, 128), jnp.float32),
    scratch_types=[
        pltpu.VMEM((RT_PER_W, 8, 128), jnp.int32),
        pltpu.VMEM((RT_PER_W, V_TILES, 8, 128), jnp.float32),
    ],
    compiler_params=pltpu.CompilerParams(
        needs_layout_passes=False,
        disable_bounds_checks=True,
        skip_device_barrier=True,
    ),
)
def _bag_of_words(in_hbm, out_hbm, tok_v, acc_v):
    wid = lax.axis_index("s") * NC + lax.axis_index("c")

    pltpu.sync_copy(in_hbm.at[pl.ds(wid * RT_PER_W, RT_PER_W)], tok_v)

    lane = lax.iota(jnp.int32, LANES)
    zf = jnp.zeros((LANES,), jnp.float32)
    ones = jnp.ones((LANES,), jnp.float32)

    def zero_tile(i, carry):
        a = i // V_TILES
        b = i % V_TILES
        for r in range(8):
            for k in range(8):
                acc_v[a, b, r, pl.ds(k * LANES, LANES)] = zf
        return carry

    lax.fori_loop(0, RT_PER_W * V_TILES, zero_tile, 0)

    for g in range(GROUPS):
        row_idx = lane + (g * LANES)
        rt = row_idx >> 3
        rs = row_idx & 7
        for l in range(L):
            col_l = jnp.full((LANES,), l, jnp.int32)
            tok = plsc.load_gather(tok_v, [rt, rs, col_l])
            m = tok >= 1
            col = jnp.maximum(tok - 1, 0)
            plsc.addupdate_scatter(
                acc_v, [rt, col >> 7, rs, col & 127], ones, mask=m
            )

    pltpu.sync_copy(acc_v, out_hbm.at[pl.ds(wid * RT_PER_W, RT_PER_W)])


def kernel(inputs):
    padded = jnp.pad(inputs, ((0, 0), (0, 128 - L)))
    out4 = _bag_of_words(padded.reshape(B // 8, 8, 128))
    out = out4.transpose(0, 2, 1, 3).reshape(B, V_PAD)
    return out[:, :OUT_V]


# speedup vs baseline: 1.0160x; 1.0144x over previous
import functools

import jax
import jax.numpy as jnp
from jax import lax
from jax.experimental import pallas as pl
from jax.experimental.pallas import tpu as pltpu
from jax.experimental.pallas import tpu_sc as plsc

B = 1024
L = 50
OUT_V = 1099
V_TILES = 9
V_PAD = V_TILES * 128
LANES = 16
NC = 2
NS = 16
NW = NC * NS
ROWS_PER_W = B // NW
RT_PER_W = ROWS_PER_W // 8
GROUPS = ROWS_PER_W // LANES


@functools.partial(
    pl.kernel,
    mesh=plsc.VectorSubcoreMesh(core_axis_name="c", subcore_axis_name="s"),
    out_type=jax.ShapeDtypeStruct((B // 8, 8, V_PAD), jnp.float32),
    scratch_types=[
        pltpu.VMEM((RT_PER_W, 8, 128), jnp.int32),
        pltpu.VMEM((RT_PER_W, V_TILES, 8, 128), jnp.float32),
        pltpu.SemaphoreType.DMA,
    ],
    compiler_params=pltpu.CompilerParams(
        needs_layout_passes=False,
        disable_bounds_checks=True,
        skip_device_barrier=True,
    ),
)
def _bag_of_words(in_hbm, out_hbm, tok_v, acc_v, sem):
    wid = lax.axis_index("s") * NC + lax.axis_index("c")

    pltpu.sync_copy(in_hbm.at[pl.ds(wid * RT_PER_W, RT_PER_W)], tok_v)

    lane = lax.iota(jnp.int32, LANES)
    zf = jnp.zeros((LANES,), jnp.float32)
    ones = jnp.ones((LANES,), jnp.float32)

    def zero_rt(a, carry):
        def zero_vt(b, carry2):
            for r in range(8):
                for k in range(8):
                    acc_v[a, b, r, pl.ds(k * LANES, LANES)] = zf
            return carry2

        lax.fori_loop(0, V_TILES, zero_vt, 0)
        return carry

    lax.fori_loop(0, RT_PER_W, zero_rt, 0)

    copies = []
    for g in range(GROUPS):
        row_idx = lane + (g * LANES)
        rt = row_idx >> 3
        rs = row_idx & 7
        for l in range(L):
            col_l = jnp.full((LANES,), l, jnp.int32)
            tok = plsc.load_gather(tok_v, [rt, rs, col_l])
            m = tok >= 1
            col = jnp.maximum(tok - 1, 0)
            plsc.addupdate_scatter(
                acc_v, [rt, col >> 7, rs, col & 127], ones, mask=m
            )
        for t in range(V_TILES):
            copies.append(
                pltpu.async_copy(
                    acc_v.at[pl.ds(g * 2, 2), t],
                    out_hbm.at[pl.ds(wid * RT_PER_W + g * 2, 2), :,
                               pl.ds(t * 128, 128)],
                    sem,
                )
            )
    for c in copies:
        c.wait()


def kernel(inputs):
    padded = jnp.pad(inputs, ((0, 0), (0, 128 - L)))
    out3 = _bag_of_words(padded.reshape(B // 8, 8, 128))
    return out3.reshape(B, V_PAD)[:, :OUT_V]
